# Initial kernel scaffold; baseline (speedup 1.0000x reference)
import jax
import jax.numpy as jnp
from jax.experimental import pallas as pl


def kernel(x, W0l, W0r, b0, w_gate, eWl, eWr, eb, W2l, W2r, b2, W3l, W3r, b3, edge_index):
    raise NotImplementedError("write your pallas kernel here")


# speedup vs baseline: 4.3025x; 4.3025x over previous
import functools

import jax
import jax.numpy as jnp
from jax import lax
from jax.experimental import pallas as pl
from jax.experimental.pallas import tpu as pltpu
from jax.experimental.pallas import tpu_sc as plsc

N = 10000
E = 320000
D = 128
NUM_EXPERTS = 8

NC = 2
NS = 16
CHUNK = 80
EPC = E // NC
EPT = EPC // NS
NCHUNK = EPT // CHUNK
RPT = 624
REM = N - NS * RPT
ZR = 208

B = 1000
GRID = N // B

def _dot(a, b):
    return jnp.dot(a, b, preferred_element_type=jnp.float32)


def _fill_f32(ref, nrow, ncol, value):
    ngrp = ncol // 16
    def body(i, carry):
        r = i // ngrp
        g = i % ngrp
        ref[r, pl.ds(g * 16, 16)] = jnp.full((16,), value, jnp.float32)
        return carry
    lax.fori_loop(0, nrow * ngrp, body, 0)


_SC_MESH = plsc.VectorSubcoreMesh(core_axis_name="c", subcore_axis_name="s")


@functools.partial(
    pl.kernel, mesh=_SC_MESH,
    out_type=[jax.ShapeDtypeStruct((NC, N, D), jnp.float32)],
    scratch_types=[
        pltpu.VMEM((CHUNK,), jnp.int32),
        pltpu.VMEM((CHUNK,), jnp.int32),
        pltpu.VMEM((CHUNK, D), jnp.float32),
        pltpu.VMEM((ZR, D), jnp.float32),
        pltpu.VMEM_SHARED((N, D), jnp.float32),
        pltpu.SemaphoreType.DMA,
    ])
def _sc_agg(h_hbm, src_hbm, dst_hbm, agg_hbm,
            src_v, dst_v, rows_v, zrow_v, agg_sh, sem):
    c = lax.axis_index("c")
    s = lax.axis_index("s")

    _fill_f32(zrow_v, ZR, D, 0.0)
    r0 = s * RPT
    for j in range(RPT // ZR):
        pltpu.sync_copy(zrow_v, agg_sh.at[pl.ds(r0 + j * ZR, ZR)])

    @pl.when(s == NS - 1)
    def _zero_rem():
        pltpu.sync_copy(zrow_v.at[pl.ds(0, REM)],
                        agg_sh.at[pl.ds(NS * RPT, REM)])

    plsc.subcore_barrier()

    ebase = c * EPC + s * EPT

    def chunk(i, carry):
        off = pl.multiple_of(ebase + i * CHUNK, 8)
        pltpu.sync_copy(src_hbm.at[pl.ds(off, CHUNK)], src_v)
        pltpu.sync_copy(dst_hbm.at[pl.ds(off, CHUNK)], dst_v)
        pltpu.async_copy(h_hbm.at[src_v], rows_v, sem).wait()
        pltpu.sync_copy(rows_v, agg_sh.at[dst_v], add=True)
        return carry

    lax.fori_loop(0, NCHUNK, chunk, 0)
    plsc.subcore_barrier()

    pltpu.sync_copy(agg_sh.at[pl.ds(r0, RPT)], agg_hbm.at[c, pl.ds(r0, RPT)])

    @pl.when(s == NS - 1)
    def _write_rem():
        pltpu.sync_copy(agg_sh.at[pl.ds(NS * RPT, REM)],
                        agg_hbm.at[c, pl.ds(NS * RPT, REM)])


@functools.partial(
    pl.kernel, mesh=_SC_MESH,
    out_type=[jax.ShapeDtypeStruct((NC, N, D), jnp.float32)],
    scratch_types=[
        pltpu.VMEM((CHUNK,), jnp.int32),
        pltpu.VMEM((CHUNK, D), jnp.float32),
        pltpu.VMEM((ZR, D), jnp.float32),
        pltpu.VMEM_SHARED((N, D), jnp.float32),
    ])
def _sc_cnt(dst_hbm, cnt_hbm, dst_v, ones_v, zcnt_v, cnt_sh):
    c = lax.axis_index("c")
    s = lax.axis_index("s")

    _fill_f32(zcnt_v, ZR, D, 0.0)
    _fill_f32(ones_v, CHUNK, D, 1.0)
    r0 = s * RPT
    for j in range(RPT // ZR):
        pltpu.sync_copy(zcnt_v, cnt_sh.at[pl.ds(r0 + j * ZR, ZR)])

    @pl.when(s == NS - 1)
    def _zero_rem():
        pltpu.sync_copy(zcnt_v.at[pl.ds(0, REM)],
                        cnt_sh.at[pl.ds(NS * RPT, REM)])

    plsc.subcore_barrier()

    ebase = c * EPC + s * EPT

    def chunk(i, carry):
        off = pl.multiple_of(ebase + i * CHUNK, 8)
        pltpu.sync_copy(dst_hbm.at[pl.ds(off, CHUNK)], dst_v)
        pltpu.sync_copy(ones_v, cnt_sh.at[dst_v], add=True)
        return carry

    lax.fori_loop(0, NCHUNK, chunk, 0)
    plsc.subcore_barrier()

    pltpu.sync_copy(cnt_sh.at[pl.ds(r0, RPT)], cnt_hbm.at[c, pl.ds(r0, RPT)])

    @pl.when(s == NS - 1)
    def _write_rem():
        pltpu.sync_copy(cnt_sh.at[pl.ds(NS * RPT, REM)],
                        cnt_hbm.at[c, pl.ds(NS * RPT, REM)])


def _sage_first_body(agg_ref, cnt_ref, x_ref, wl_ref, wr_ref, b_ref,
                     h_ref, cm_ref):
    a = agg_ref[0] + agg_ref[1]
    cm = jnp.maximum(cnt_ref[0, :, 0:16] + cnt_ref[1, :, 0:16], 1.0)
    mean = a / cm[:, 0:1]
    out = _dot(mean, wl_ref[...]) + _dot(x_ref[...], wr_ref[...]) + b_ref[...]
    h_ref[...] = jnp.maximum(out, 0.0)
    cm_ref[...] = cm


def _sage_body(relu, agg_ref, cm_ref, x_ref, wl_ref, wr_ref, b_ref, o_ref):
    mean = (agg_ref[0] + agg_ref[1]) / cm_ref[:, 0:1]
    out = _dot(mean, wl_ref[...]) + _dot(x_ref[...], wr_ref[...]) + b_ref[...]
    o_ref[...] = jnp.maximum(out, 0.0) if relu else out


def _moe_body(agg_ref, cm_ref, h_ref, wg_ref, ewl_ref, ewr_ref, eb_ref,
              o_ref):
    h = h_ref[...]
    mean = (agg_ref[0] + agg_ref[1]) / cm_ref[:, 0:1]
    logits = _dot(h, wg_ref[...])
    eidx = lax.broadcasted_iota(jnp.int32, logits.shape, 1)
    m1 = jnp.max(logits, axis=1, keepdims=True)
    i1 = jnp.min(jnp.where(logits == m1, eidx, NUM_EXPERTS), axis=1,
                 keepdims=True)
    rest = jnp.where(eidx == i1, -jnp.inf, logits)
    m2 = jnp.max(rest, axis=1, keepdims=True)
    i2 = jnp.min(jnp.where(rest == m2, eidx, NUM_EXPERTS), axis=1,
                 keepdims=True)
    t = jnp.exp(m2 - m1)
    g1 = 1.0 / (1.0 + t)
    g2 = t / (1.0 + t)
    acc = jnp.zeros_like(h)
    for e in range(NUM_EXPERTS):
        ge = jnp.where(i1 == e, g1, 0.0) + jnp.where(i2 == e, g2, 0.0)
        pe = _dot(mean, ewl_ref[e]) + _dot(h, ewr_ref[e]) + eb_ref[e:e + 1, :]
        acc = acc + ge * pe
    o_ref[...] = jnp.maximum(acc, 0.0)


def _row_spec(shape3=False, width=D):
    if shape3:
        return pl.BlockSpec((NC, B, width), lambda i: (0, i, 0))
    return pl.BlockSpec((B, width), lambda i: (i, 0))


def _full_spec(shape):
    nd = len(shape)
    return pl.BlockSpec(shape, lambda i: (0,) * nd)


def _tc_sage_first(agg, cnt, x, Wl, Wr, b):
    return pl.pallas_call(
        _sage_first_body,
        grid=(GRID,),
        in_specs=[
            _row_spec(True), _row_spec(True), _row_spec(),
            _full_spec((D, D)), _full_spec((D, D)), _full_spec((1, D)),
        ],
        out_specs=[_row_spec(), _row_spec(False, 16)],
        out_shape=[jax.ShapeDtypeStruct((N, D), jnp.float32),
                   jax.ShapeDtypeStruct((N, 16), jnp.float32)],
    )(agg, cnt, x, Wl, Wr, b.reshape(1, D))


def _tc_sage(agg, cm, x, Wl, Wr, b, relu):
    return pl.pallas_call(
        functools.partial(_sage_body, relu),
        grid=(GRID,),
        in_specs=[
            _row_spec(True), _row_spec(False, 16), _row_spec(),
            _full_spec((D, D)), _full_spec((D, D)), _full_spec((1, D)),
        ],
        out_specs=_row_spec(),
        out_shape=jax.ShapeDtypeStruct((N, D), jnp.float32),
    )(agg, cm, x, Wl, Wr, b.reshape(1, D))


def _tc_moe(agg, cm, h, w_gate, eWl, eWr, eb):
    return pl.pallas_call(
        _moe_body,
        grid=(GRID,),
        in_specs=[
            _row_spec(True), _row_spec(False, 16), _row_spec(),
            _full_spec((D, NUM_EXPERTS)),
            _full_spec((NUM_EXPERTS, D, D)), _full_spec((NUM_EXPERTS, D, D)),
            _full_spec((NUM_EXPERTS, D)),
        ],
        out_specs=_row_spec(),
        out_shape=jax.ShapeDtypeStruct((N, D), jnp.float32),
    )(agg, cm, h, w_gate, eWl, eWr, eb)


def kernel(x, W0l, W0r, b0, w_gate, eWl, eWr, eb, W2l, W2r, b2, W3l, W3r, b3,
           edge_index):
    src = edge_index[0].astype(jnp.int32)
    dst = edge_index[1].astype(jnp.int32)

    (cnt0,) = _sc_cnt(dst)
    (agg0,) = _sc_agg(x, src, dst)
    h1, cm = _tc_sage_first(agg0, cnt0, x, W0l, W0r, b0)
    (agg1,) = _sc_agg(h1, src, dst)
    h2 = _tc_moe(agg1, cm, h1, w_gate, eWl, eWr, eb)
    (agg2,) = _sc_agg(h2, src, dst)
    h3 = _tc_sage(agg2, cm, h2, W2l, W2r, b2, relu=True)
    (agg3,) = _sc_agg(h3, src, dst)
    out = _tc_sage(agg3, cm, h3, W3l, W3r, b3, relu=False)
    return out


# speedup vs baseline: 8.6327x; 2.0064x over previous
import functools

import jax
import jax.numpy as jnp
from jax import lax
from jax.experimental import pallas as pl
from jax.experimental.pallas import tpu as pltpu
from jax.experimental.pallas import tpu_sc as plsc

N = 10000
E = 320000
D = 128
NUM_EXPERTS = 8

NC = 2
NS = 16
CHUNK = 40
EPC = E // NC
EPT = EPC // NS
NCHUNK = EPT // CHUNK
RPT = 624
REM = N - NS * RPT

B = 1000
GRID = N // B

NBUF = 6
STEADY0 = NBUF
STEADYN = ((NCHUNK - 4) // NBUF) * NBUF


def _dot(a, b):
    return jnp.dot(a, b, preferred_element_type=jnp.float32)


def _fill_f32(ref, nrow, ncol, value):
    ngrp = ncol // 16
    def body(i, carry):
        r = i // ngrp
        g = i % ngrp
        ref[r, pl.ds(g * 16, 16)] = jnp.full((16,), value, jnp.float32)
        return carry
    lax.fori_loop(0, nrow * ngrp, body, 0)


_SC_MESH = plsc.VectorSubcoreMesh(core_axis_name="c", subcore_axis_name="s")


@functools.partial(
    pl.kernel, mesh=_SC_MESH,
    out_type=[jax.ShapeDtypeStruct((NC, N, D), jnp.float32)],
    scratch_types=(
        [pltpu.VMEM((CHUNK,), jnp.int32) for _ in range(NBUF)] +
        [pltpu.VMEM((CHUNK,), jnp.int32) for _ in range(NBUF)] +
        [pltpu.VMEM((CHUNK, D), jnp.float32) for _ in range(NBUF)] +
        [pltpu.VMEM_SHARED((N, D), jnp.float32)] +
        [pltpu.SemaphoreType.DMA for _ in range(3 * NBUF)]
    ))
def _sc_agg(h_hbm, src_hbm, dst_hbm, zer_hbm, agg_hbm, *scr):
    c = lax.axis_index("c")
    s = lax.axis_index("s")
    sidx = list(scr[0:NBUF])
    didx = list(scr[NBUF:2 * NBUF])
    rows = list(scr[2 * NBUF:3 * NBUF])
    agg_sh = scr[3 * NBUF]
    isem = list(scr[3 * NBUF + 1:4 * NBUF + 1])
    gsem = list(scr[4 * NBUF + 1:5 * NBUF + 1])
    ssem = list(scr[5 * NBUF + 1:6 * NBUF + 1])
    wid = c * NS + s
    ebase = wid * EPT

    def i_start(t, j):
        off = pl.multiple_of(ebase + t * CHUNK, 8)
        pltpu.async_copy(src_hbm.at[pl.ds(off, CHUNK)], sidx[j], isem[j])
        pltpu.async_copy(dst_hbm.at[pl.ds(off, CHUNK)], didx[j], isem[j])

    def i_wait(t, j):
        off = pl.multiple_of(ebase + t * CHUNK, 8)
        pltpu.make_async_copy(src_hbm.at[pl.ds(off, CHUNK)], sidx[j],
                              isem[j]).wait()
        pltpu.make_async_copy(dst_hbm.at[pl.ds(off, CHUNK)], didx[j],
                              isem[j]).wait()

    def g_start(t, j):
        pltpu.async_copy(h_hbm.at[sidx[j]], rows[j], gsem[j])

    def g_wait(t, j):
        pltpu.make_async_copy(h_hbm.at[sidx[j]], rows[j], gsem[j]).wait()

    def s_start(t, j):
        pltpu.async_copy(rows[j], agg_sh.at[didx[j]], ssem[j], add=True)

    def s_wait(t, j):
        pltpu.make_async_copy(rows[j], agg_sh.at[didx[j]], ssem[j]).wait()

    r0 = s * RPT
    pltpu.sync_copy(zer_hbm, agg_sh.at[pl.ds(r0, RPT)])

    @pl.when(s == NS - 1)
    def _zero_rem():
        pltpu.sync_copy(zer_hbm.at[pl.ds(0, REM)],
                        agg_sh.at[pl.ds(NS * RPT, REM)])

    plsc.subcore_barrier()

    def step(t, j, steady):
        g_wait(t, j)
        s_start(t, j)
        if steady or t >= 2:
            s_wait(t - 2, (j + 4) % NBUF)
        if steady or t + 4 < NCHUNK:
            i_start(t + 4, (j + 4) % NBUF)
        if steady or t + 2 < NCHUNK:
            i_wait(t + 2, (j + 2) % NBUF)
            g_start(t + 2, (j + 2) % NBUF)

    for t in range(4):
        i_start(t, t)
    for t in range(2):
        i_wait(t, t)
        g_start(t, t)

    for t in range(STEADY0):
        step(t, t % NBUF, False)

    def body(k, carry):
        for jj in range(NBUF):
            step(k * NBUF + jj, jj, True)
        return carry

    lax.fori_loop(1, STEADYN // NBUF, body, 0)

    for t in range(STEADYN, NCHUNK):
        step(t, t % NBUF, False)

    for t in range(NCHUNK - 2, NCHUNK):
        s_wait(t, t % NBUF)

    plsc.subcore_barrier()

    pltpu.sync_copy(agg_sh.at[pl.ds(r0, RPT)], agg_hbm.at[c, pl.ds(r0, RPT)])

    @pl.when(s == NS - 1)
    def _write_rem():
        pltpu.sync_copy(agg_sh.at[pl.ds(NS * RPT, REM)],
                        agg_hbm.at[c, pl.ds(NS * RPT, REM)])


@functools.partial(
    pl.kernel, mesh=_SC_MESH,
    out_type=[jax.ShapeDtypeStruct((NC, N, D), jnp.float32)],
    scratch_types=(
        [pltpu.VMEM((CHUNK,), jnp.int32) for _ in range(NBUF)] +
        [pltpu.VMEM((CHUNK, D), jnp.float32)] +
        [pltpu.VMEM_SHARED((N, D), jnp.float32)] +
        [pltpu.SemaphoreType.DMA for _ in range(2 * NBUF)]
    ))
def _sc_cnt(dst_hbm, zer_hbm, cnt_hbm, *scr):
    c = lax.axis_index("c")
    s = lax.axis_index("s")
    didx = list(scr[0:NBUF])
    ones_v = scr[NBUF]
    cnt_sh = scr[NBUF + 1]
    isem = list(scr[NBUF + 2:2 * NBUF + 2])
    ssem = list(scr[2 * NBUF + 2:3 * NBUF + 2])
    wid = c * NS + s
    ebase = wid * EPT

    def i_start(t, j):
        off = pl.multiple_of(ebase + t * CHUNK, 8)
        pltpu.async_copy(dst_hbm.at[pl.ds(off, CHUNK)], didx[j], isem[j])

    def i_wait(t, j):
        off = pl.multiple_of(ebase + t * CHUNK, 8)
        pltpu.make_async_copy(dst_hbm.at[pl.ds(off, CHUNK)], didx[j],
                              isem[j]).wait()

    def s_start(t, j):
        pltpu.async_copy(ones_v, cnt_sh.at[didx[j]], ssem[j], add=True)

    def s_wait(t, j):
        pltpu.make_async_copy(ones_v, cnt_sh.at[didx[j]], ssem[j]).wait()

    _fill_f32(ones_v, CHUNK, D, 1.0)
    r0 = s * RPT
    pltpu.sync_copy(zer_hbm, cnt_sh.at[pl.ds(r0, RPT)])

    @pl.when(s == NS - 1)
    def _zero_rem():
        pltpu.sync_copy(zer_hbm.at[pl.ds(0, REM)],
                        cnt_sh.at[pl.ds(NS * RPT, REM)])

    plsc.subcore_barrier()

    def step(t, j, steady):
        i_wait(t, j)
        s_start(t, j)
        if steady or t >= 2:
            s_wait(t - 2, (j + 4) % NBUF)
        if steady or t + 4 < NCHUNK:
            i_start(t + 4, (j + 4) % NBUF)

    for t in range(4):
        i_start(t, t)

    for t in range(STEADY0):
        step(t, t % NBUF, False)

    def body(k, carry):
        for jj in range(NBUF):
            step(k * NBUF + jj, jj, True)
        return carry

    lax.fori_loop(1, STEADYN // NBUF, body, 0)

    for t in range(STEADYN, NCHUNK):
        step(t, t % NBUF, False)

    for t in range(NCHUNK - 2, NCHUNK):
        s_wait(t, t % NBUF)

    plsc.subcore_barrier()

    pltpu.sync_copy(cnt_sh.at[pl.ds(r0, RPT)], cnt_hbm.at[c, pl.ds(r0, RPT)])

    @pl.when(s == NS - 1)
    def _write_rem():
        pltpu.sync_copy(cnt_sh.at[pl.ds(NS * RPT, REM)],
                        cnt_hbm.at[c, pl.ds(NS * RPT, REM)])


def _sage_first_body(agg_ref, cnt_ref, x_ref, wl_ref, wr_ref, b_ref,
                     h_ref, cm_ref):
    a = agg_ref[0] + agg_ref[1]
    cm = jnp.maximum(cnt_ref[0, :, 0:16] + cnt_ref[1, :, 0:16], 1.0)
    mean = a / cm[:, 0:1]
    out = _dot(mean, wl_ref[...]) + _dot(x_ref[...], wr_ref[...]) + b_ref[...]
    h_ref[...] = jnp.maximum(out, 0.0)
    cm_ref[...] = cm


def _sage_body(relu, agg_ref, cm_ref, x_ref, wl_ref, wr_ref, b_ref, o_ref):
    mean = (agg_ref[0] + agg_ref[1]) / cm_ref[:, 0:1]
    out = _dot(mean, wl_ref[...]) + _dot(x_ref[...], wr_ref[...]) + b_ref[...]
    o_ref[...] = jnp.maximum(out, 0.0) if relu else out


def _moe_body(agg_ref, cm_ref, h_ref, wg_ref, ewl_ref, ewr_ref, eb_ref,
              o_ref):
    h = h_ref[...]
    mean = (agg_ref[0] + agg_ref[1]) / cm_ref[:, 0:1]
    logits = _dot(h, wg_ref[...])
    eidx = lax.broadcasted_iota(jnp.int32, logits.shape, 1)
    m1 = jnp.max(logits, axis=1, keepdims=True)
    i1 = jnp.min(jnp.where(logits == m1, eidx, NUM_EXPERTS), axis=1,
                 keepdims=True)
    rest = jnp.where(eidx == i1, -jnp.inf, logits)
    m2 = jnp.max(rest, axis=1, keepdims=True)
    i2 = jnp.min(jnp.where(rest == m2, eidx, NUM_EXPERTS), axis=1,
                 keepdims=True)
    t = jnp.exp(m2 - m1)
    g1 = 1.0 / (1.0 + t)
    g2 = t / (1.0 + t)
    acc = jnp.zeros_like(h)
    for e in range(NUM_EXPERTS):
        ge = jnp.where(i1 == e, g1, 0.0) + jnp.where(i2 == e, g2, 0.0)
        pe = _dot(mean, ewl_ref[e]) + _dot(h, ewr_ref[e]) + eb_ref[e:e + 1, :]
        acc = acc + ge * pe
    o_ref[...] = jnp.maximum(acc, 0.0)


def _row_spec(shape3=False, width=D):
    if shape3:
        return pl.BlockSpec((NC, B, width), lambda i: (0, i, 0))
    return pl.BlockSpec((B, width), lambda i: (i, 0))


def _full_spec(shape):
    nd = len(shape)
    return pl.BlockSpec(shape, lambda i: (0,) * nd)


def _tc_sage_first(agg, cnt, x, Wl, Wr, b):
    return pl.pallas_call(
        _sage_first_body,
        grid=(GRID,),
        in_specs=[
            _row_spec(True), _row_spec(True), _row_spec(),
            _full_spec((D, D)), _full_spec((D, D)), _full_spec((1, D)),
        ],
        out_specs=[_row_spec(), _row_spec(False, 16)],
        out_shape=[jax.ShapeDtypeStruct((N, D), jnp.float32),
                   jax.ShapeDtypeStruct((N, 16), jnp.float32)],
    )(agg, cnt, x, Wl, Wr, b.reshape(1, D))


def _tc_sage(agg, cm, x, Wl, Wr, b, relu):
    return pl.pallas_call(
        functools.partial(_sage_body, relu),
        grid=(GRID,),
        in_specs=[
            _row_spec(True), _row_spec(False, 16), _row_spec(),
            _full_spec((D, D)), _full_spec((D, D)), _full_spec((1, D)),
        ],
        out_specs=_row_spec(),
        out_shape=jax.ShapeDtypeStruct((N, D), jnp.float32),
    )(agg, cm, x, Wl, Wr, b.reshape(1, D))


def _tc_moe(agg, cm, h, w_gate, eWl, eWr, eb):
    return pl.pallas_call(
        _moe_body,
        grid=(GRID,),
        in_specs=[
            _row_spec(True), _row_spec(False, 16), _row_spec(),
            _full_spec((D, NUM_EXPERTS)),
            _full_spec((NUM_EXPERTS, D, D)), _full_spec((NUM_EXPERTS, D, D)),
            _full_spec((NUM_EXPERTS, D)),
        ],
        out_specs=_row_spec(),
        out_shape=jax.ShapeDtypeStruct((N, D), jnp.float32),
    )(agg, cm, h, w_gate, eWl, eWr, eb)


def kernel(x, W0l, W0r, b0, w_gate, eWl, eWr, eb, W2l, W2r, b2, W3l, W3r, b3,
           edge_index):
    src = edge_index[0].astype(jnp.int32)
    dst = edge_index[1].astype(jnp.int32)
    zer = jnp.zeros((RPT, D), jnp.float32)

    (cnt0,) = _sc_cnt(dst, zer)
    (agg0,) = _sc_agg(x, src, dst, zer)
    h1, cm = _tc_sage_first(agg0, cnt0, x, W0l, W0r, b0)
    (agg1,) = _sc_agg(h1, src, dst, zer)
    h2 = _tc_moe(agg1, cm, h1, w_gate, eWl, eWr, eb)
    (agg2,) = _sc_agg(h2, src, dst, zer)
    h3 = _tc_sage(agg2, cm, h2, W2l, W2r, b2, relu=True)
    (agg3,) = _sc_agg(h3, src, dst, zer)
    out = _tc_sage(agg3, cm, h3, W3l, W3r, b3, relu=False)
    return out


# speedup vs baseline: 11.1556x; 1.2923x over previous
import functools

import jax
import jax.numpy as jnp
from jax import lax
from jax.experimental import pallas as pl
from jax.experimental.pallas import tpu as pltpu
from jax.experimental.pallas import tpu_sc as plsc

N = 10000
E = 320000
D = 128
NUM_EXPERTS = 8

NC = 2
NS = 16
CHUNK = 40
EPC = E // NC
EPT = EPC // NS
NCHUNK = EPT // CHUNK
RPT = 624
REM = N - NS * RPT

B = 1000
GRID = N // B

NBUF = 8
IA = 6
GA = 4
STEADY0 = NBUF
STEADYN = ((NCHUNK - IA) // NBUF) * NBUF


def _dot(a, b):
    return jnp.dot(a, b, preferred_element_type=jnp.float32)


def _fill_f32(ref, nrow, ncol, value):
    ngrp = ncol // 16
    def body(i, carry):
        r = i // ngrp
        g = i % ngrp
        ref[r, pl.ds(g * 16, 16)] = jnp.full((16,), value, jnp.float32)
        return carry
    lax.fori_loop(0, nrow * ngrp, body, 0)


_SC_MESH = plsc.VectorSubcoreMesh(core_axis_name="c", subcore_axis_name="s")


@functools.partial(
    pl.kernel, mesh=_SC_MESH,
    out_type=[jax.ShapeDtypeStruct((NC, N, D), jnp.float32)],
    scratch_types=(
        [pltpu.VMEM((CHUNK,), jnp.int32) for _ in range(NBUF)] +
        [pltpu.VMEM((CHUNK,), jnp.int32) for _ in range(NBUF)] +
        [pltpu.VMEM((CHUNK, D), jnp.float32) for _ in range(NBUF)] +
        [pltpu.VMEM_SHARED((N, D), jnp.float32)] +
        [pltpu.SemaphoreType.DMA for _ in range(3 * NBUF)]
    ))
def _sc_agg(h_hbm, src_hbm, dst_hbm, zer_hbm, agg_hbm, *scr):
    c = lax.axis_index("c")
    s = lax.axis_index("s")
    sidx = list(scr[0:NBUF])
    didx = list(scr[NBUF:2 * NBUF])
    rows = list(scr[2 * NBUF:3 * NBUF])
    agg_sh = scr[3 * NBUF]
    isem = list(scr[3 * NBUF + 1:4 * NBUF + 1])
    gsem = list(scr[4 * NBUF + 1:5 * NBUF + 1])
    ssem = list(scr[5 * NBUF + 1:6 * NBUF + 1])
    wid = c * NS + s
    ebase = wid * EPT

    def i_start(t, j):
        off = pl.multiple_of(ebase + t * CHUNK, 8)
        pltpu.async_copy(src_hbm.at[pl.ds(off, CHUNK)], sidx[j], isem[j])
        pltpu.async_copy(dst_hbm.at[pl.ds(off, CHUNK)], didx[j], isem[j])

    def i_wait(t, j):
        off = pl.multiple_of(ebase + t * CHUNK, 8)
        pltpu.make_async_copy(src_hbm.at[pl.ds(off, CHUNK)], sidx[j],
                              isem[j]).wait()
        pltpu.make_async_copy(dst_hbm.at[pl.ds(off, CHUNK)], didx[j],
                              isem[j]).wait()

    def g_start(t, j):
        pltpu.async_copy(h_hbm.at[sidx[j]], rows[j], gsem[j])

    def g_wait(t, j):
        pltpu.make_async_copy(h_hbm.at[sidx[j]], rows[j], gsem[j]).wait()

    def s_start(t, j):
        pltpu.async_copy(rows[j], agg_sh.at[didx[j]], ssem[j], add=True)

    def s_wait(t, j):
        pltpu.make_async_copy(rows[j], agg_sh.at[didx[j]], ssem[j]).wait()

    r0 = s * RPT
    pltpu.sync_copy(zer_hbm, agg_sh.at[pl.ds(r0, RPT)])

    @pl.when(s == NS - 1)
    def _zero_rem():
        pltpu.sync_copy(zer_hbm.at[pl.ds(0, REM)],
                        agg_sh.at[pl.ds(NS * RPT, REM)])

    plsc.subcore_barrier()

    def step(t, j, steady):
        g_wait(t, j)
        s_start(t, j)
        if steady or t >= 2:
            s_wait(t - 2, (j + IA) % NBUF)
        if steady or t + IA < NCHUNK:
            i_start(t + IA, (j + IA) % NBUF)
        if steady or t + GA < NCHUNK:
            i_wait(t + GA, (j + GA) % NBUF)
            g_start(t + GA, (j + GA) % NBUF)

    for t in range(IA):
        i_start(t, t)
    for t in range(GA):
        i_wait(t, t)
        g_start(t, t)

    for t in range(STEADY0):
        step(t, t % NBUF, False)

    def body(k, carry):
        for jj in range(NBUF):
            step(k * NBUF + jj, jj, True)
        return carry

    lax.fori_loop(1, STEADYN // NBUF, body, 0)

    for t in range(STEADYN, NCHUNK):
        step(t, t % NBUF, False)

    for t in range(NCHUNK - 2, NCHUNK):
        s_wait(t, t % NBUF)

    plsc.subcore_barrier()

    pltpu.sync_copy(agg_sh.at[pl.ds(r0, RPT)], agg_hbm.at[c, pl.ds(r0, RPT)])

    @pl.when(s == NS - 1)
    def _write_rem():
        pltpu.sync_copy(agg_sh.at[pl.ds(NS * RPT, REM)],
                        agg_hbm.at[c, pl.ds(NS * RPT, REM)])


@functools.partial(
    pl.kernel, mesh=_SC_MESH,
    out_type=[jax.ShapeDtypeStruct((NC, N, D), jnp.float32)],
    scratch_types=(
        [pltpu.VMEM((CHUNK,), jnp.int32) for _ in range(NBUF)] +
        [pltpu.VMEM((CHUNK, D), jnp.float32)] +
        [pltpu.VMEM_SHARED((N, D), jnp.float32)] +
        [pltpu.SemaphoreType.DMA for _ in range(2 * NBUF)]
    ))
def _sc_cnt(dst_hbm, zer_hbm, cnt_hbm, *scr):
    c = lax.axis_index("c")
    s = lax.axis_index("s")
    didx = list(scr[0:NBUF])
    ones_v = scr[NBUF]
    cnt_sh = scr[NBUF + 1]
    isem = list(scr[NBUF + 2:2 * NBUF + 2])
    ssem = list(scr[2 * NBUF + 2:3 * NBUF + 2])
    wid = c * NS + s
    ebase = wid * EPT

    def i_start(t, j):
        off = pl.multiple_of(ebase + t * CHUNK, 8)
        pltpu.async_copy(dst_hbm.at[pl.ds(off, CHUNK)], didx[j], isem[j])

    def i_wait(t, j):
        off = pl.multiple_of(ebase + t * CHUNK, 8)
        pltpu.make_async_copy(dst_hbm.at[pl.ds(off, CHUNK)], didx[j],
                              isem[j]).wait()

    def s_start(t, j):
        pltpu.async_copy(ones_v, cnt_sh.at[didx[j]], ssem[j], add=True)

    def s_wait(t, j):
        pltpu.make_async_copy(ones_v, cnt_sh.at[didx[j]], ssem[j]).wait()

    _fill_f32(ones_v, CHUNK, D, 1.0)
    r0 = s * RPT
    pltpu.sync_copy(zer_hbm, cnt_sh.at[pl.ds(r0, RPT)])

    @pl.when(s == NS - 1)
    def _zero_rem():
        pltpu.sync_copy(zer_hbm.at[pl.ds(0, REM)],
                        cnt_sh.at[pl.ds(NS * RPT, REM)])

    plsc.subcore_barrier()

    def step(t, j, steady):
        i_wait(t, j)
        s_start(t, j)
        if steady or t >= 2:
            s_wait(t - 2, (j + IA) % NBUF)
        if steady or t + IA < NCHUNK:
            i_start(t + IA, (j + IA) % NBUF)

    for t in range(IA):
        i_start(t, t)

    for t in range(STEADY0):
        step(t, t % NBUF, False)

    def body(k, carry):
        for jj in range(NBUF):
            step(k * NBUF + jj, jj, True)
        return carry

    lax.fori_loop(1, STEADYN // NBUF, body, 0)

    for t in range(STEADYN, NCHUNK):
        step(t, t % NBUF, False)

    for t in range(NCHUNK - 2, NCHUNK):
        s_wait(t, t % NBUF)

    plsc.subcore_barrier()

    pltpu.sync_copy(cnt_sh.at[pl.ds(r0, RPT)], cnt_hbm.at[c, pl.ds(r0, RPT)])

    @pl.when(s == NS - 1)
    def _write_rem():
        pltpu.sync_copy(cnt_sh.at[pl.ds(NS * RPT, REM)],
                        cnt_hbm.at[c, pl.ds(NS * RPT, REM)])


def _sage_first_body(agg_ref, cnt_ref, x_ref, wl_ref, wr_ref, b_ref,
                     h_ref, cm_ref):
    a = agg_ref[0] + agg_ref[1]
    cm = jnp.maximum(cnt_ref[0, :, 0:16] + cnt_ref[1, :, 0:16], 1.0)
    mean = a / cm[:, 0:1]
    out = _dot(mean, wl_ref[...]) + _dot(x_ref[...], wr_ref[...]) + b_ref[...]
    h_ref[...] = jnp.maximum(out, 0.0)
    cm_ref[...] = cm


def _sage_body(relu, agg_ref, cm_ref, x_ref, wl_ref, wr_ref, b_ref, o_ref):
    mean = (agg_ref[0] + agg_ref[1]) / cm_ref[:, 0:1]
    out = _dot(mean, wl_ref[...]) + _dot(x_ref[...], wr_ref[...]) + b_ref[...]
    o_ref[...] = jnp.maximum(out, 0.0) if relu else out


def _moe_body(agg_ref, cm_ref, h_ref, wg_ref, ewl_ref, ewr_ref, eb_ref,
              o_ref):
    h = h_ref[...]
    mean = (agg_ref[0] + agg_ref[1]) / cm_ref[:, 0:1]
    logits = _dot(h, wg_ref[...])
    eidx = lax.broadcasted_iota(jnp.int32, logits.shape, 1)
    m1 = jnp.max(logits, axis=1, keepdims=True)
    i1 = jnp.min(jnp.where(logits == m1, eidx, NUM_EXPERTS), axis=1,
                 keepdims=True)
    rest = jnp.where(eidx == i1, -jnp.inf, logits)
    m2 = jnp.max(rest, axis=1, keepdims=True)
    i2 = jnp.min(jnp.where(rest == m2, eidx, NUM_EXPERTS), axis=1,
                 keepdims=True)
    t = jnp.exp(m2 - m1)
    g1 = 1.0 / (1.0 + t)
    g2 = t / (1.0 + t)
    acc = jnp.zeros_like(h)
    for e in range(NUM_EXPERTS):
        ge = jnp.where(i1 == e, g1, 0.0) + jnp.where(i2 == e, g2, 0.0)
        pe = _dot(mean, ewl_ref[e]) + _dot(h, ewr_ref[e]) + eb_ref[e:e + 1, :]
        acc = acc + ge * pe
    o_ref[...] = jnp.maximum(acc, 0.0)


def _row_spec(shape3=False, width=D):
    if shape3:
        return pl.BlockSpec((NC, B, width), lambda i: (0, i, 0))
    return pl.BlockSpec((B, width), lambda i: (i, 0))


def _full_spec(shape):
    nd = len(shape)
    return pl.BlockSpec(shape, lambda i: (0,) * nd)


def _tc_sage_first(agg, cnt, x, Wl, Wr, b):
    return pl.pallas_call(
        _sage_first_body,
        grid=(GRID,),
        in_specs=[
            _row_spec(True), _row_spec(True), _row_spec(),
            _full_spec((D, D)), _full_spec((D, D)), _full_spec((1, D)),
        ],
        out_specs=[_row_spec(), _row_spec(False, 16)],
        out_shape=[jax.ShapeDtypeStruct((N, D), jnp.float32),
                   jax.ShapeDtypeStruct((N, 16), jnp.float32)],
    )(agg, cnt, x, Wl, Wr, b.reshape(1, D))


def _tc_sage(agg, cm, x, Wl, Wr, b, relu):
    return pl.pallas_call(
        functools.partial(_sage_body, relu),
        grid=(GRID,),
        in_specs=[
            _row_spec(True), _row_spec(False, 16), _row_spec(),
            _full_spec((D, D)), _full_spec((D, D)), _full_spec((1, D)),
        ],
        out_specs=_row_spec(),
        out_shape=jax.ShapeDtypeStruct((N, D), jnp.float32),
    )(agg, cm, x, Wl, Wr, b.reshape(1, D))


def _tc_moe(agg, cm, h, w_gate, eWl, eWr, eb):
    return pl.pallas_call(
        _moe_body,
        grid=(GRID,),
        in_specs=[
            _row_spec(True), _row_spec(False, 16), _row_spec(),
            _full_spec((D, NUM_EXPERTS)),
            _full_spec((NUM_EXPERTS, D, D)), _full_spec((NUM_EXPERTS, D, D)),
            _full_spec((NUM_EXPERTS, D)),
        ],
        out_specs=_row_spec(),
        out_shape=jax.ShapeDtypeStruct((N, D), jnp.float32),
    )(agg, cm, h, w_gate, eWl, eWr, eb)


def kernel(x, W0l, W0r, b0, w_gate, eWl, eWr, eb, W2l, W2r, b2, W3l, W3r, b3,
           edge_index):
    src = edge_index[0].astype(jnp.int32)
    dst = edge_index[1].astype(jnp.int32)
    zer = jnp.zeros((RPT, D), jnp.float32)

    (cnt0,) = _sc_cnt(dst, zer)
    (agg0,) = _sc_agg(x, src, dst, zer)
    h1, cm = _tc_sage_first(agg0, cnt0, x, W0l, W0r, b0)
    (agg1,) = _sc_agg(h1, src, dst, zer)
    h2 = _tc_moe(agg1, cm, h1, w_gate, eWl, eWr, eb)
    (agg2,) = _sc_agg(h2, src, dst, zer)
    h3 = _tc_sage(agg2, cm, h2, W2l, W2r, b2, relu=True)
    (agg3,) = _sc_agg(h3, src, dst, zer)
    out = _tc_sage(agg3, cm, h3, W3l, W3r, b3, relu=False)
    return out


# speedup vs baseline: 11.4052x; 1.0224x over previous
import functools

import jax
import jax.numpy as jnp
from jax import lax
from jax.experimental import pallas as pl
from jax.experimental.pallas import tpu as pltpu
from jax.experimental.pallas import tpu_sc as plsc

N = 10000
E = 320000
D = 128
NUM_EXPERTS = 8

NC = 2
NS = 16
CHUNK = 40
EPC = E // NC
EPT = EPC // NS
NCHUNK = EPT // CHUNK
RPT = 624
REM = N - NS * RPT

B = 2000
GRID = N // B

NBUF = 8
IA = 6
GA = 4
STEADY0 = NBUF
STEADYN = ((NCHUNK - IA) // NBUF) * NBUF


def _dot(a, b):
    return jnp.dot(a, b, preferred_element_type=jnp.float32)


def _fill_f32(ref, nrow, ncol, value):
    ngrp = ncol // 16
    def body(i, carry):
        r = i // ngrp
        g = i % ngrp
        ref[r, pl.ds(g * 16, 16)] = jnp.full((16,), value, jnp.float32)
        return carry
    lax.fori_loop(0, nrow * ngrp, body, 0)


_SC_MESH = plsc.VectorSubcoreMesh(core_axis_name="c", subcore_axis_name="s")


@functools.partial(
    pl.kernel, mesh=_SC_MESH,
    out_type=[jax.ShapeDtypeStruct((NC, N, D), jnp.float32)],
    scratch_types=(
        [pltpu.VMEM((CHUNK,), jnp.int32) for _ in range(NBUF)] +
        [pltpu.VMEM((CHUNK,), jnp.int32) for _ in range(NBUF)] +
        [pltpu.VMEM((CHUNK, D), jnp.float32) for _ in range(NBUF)] +
        [pltpu.VMEM_SHARED((N, D), jnp.float32)] +
        [pltpu.SemaphoreType.DMA for _ in range(3 * NBUF)]
    ))
def _sc_agg(h_hbm, src_hbm, dst_hbm, zer_hbm, agg_hbm, *scr):
    c = lax.axis_index("c")
    s = lax.axis_index("s")
    sidx = list(scr[0:NBUF])
    didx = list(scr[NBUF:2 * NBUF])
    rows = list(scr[2 * NBUF:3 * NBUF])
    agg_sh = scr[3 * NBUF]
    isem = list(scr[3 * NBUF + 1:4 * NBUF + 1])
    gsem = list(scr[4 * NBUF + 1:5 * NBUF + 1])
    ssem = list(scr[5 * NBUF + 1:6 * NBUF + 1])
    wid = c * NS + s
    ebase = wid * EPT

    def i_start(t, j):
        off = pl.multiple_of(ebase + t * CHUNK, 8)
        pltpu.async_copy(src_hbm.at[pl.ds(off, CHUNK)], sidx[j], isem[j])
        pltpu.async_copy(dst_hbm.at[pl.ds(off, CHUNK)], didx[j], isem[j])

    def i_wait(t, j):
        off = pl.multiple_of(ebase + t * CHUNK, 8)
        pltpu.make_async_copy(src_hbm.at[pl.ds(off, CHUNK)], sidx[j],
                              isem[j]).wait()
        pltpu.make_async_copy(dst_hbm.at[pl.ds(off, CHUNK)], didx[j],
                              isem[j]).wait()

    def g_start(t, j):
        pltpu.async_copy(h_hbm.at[sidx[j]], rows[j], gsem[j])

    def g_wait(t, j):
        pltpu.make_async_copy(h_hbm.at[sidx[j]], rows[j], gsem[j]).wait()

    def s_start(t, j):
        pltpu.async_copy(rows[j], agg_sh.at[didx[j]], ssem[j], add=True)

    def s_wait(t, j):
        pltpu.make_async_copy(rows[j], agg_sh.at[didx[j]], ssem[j]).wait()

    r0 = s * RPT
    pltpu.sync_copy(zer_hbm, agg_sh.at[pl.ds(r0, RPT)])

    @pl.when(s == NS - 1)
    def _zero_rem():
        pltpu.sync_copy(zer_hbm.at[pl.ds(0, REM)],
                        agg_sh.at[pl.ds(NS * RPT, REM)])

    plsc.subcore_barrier()

    def step(t, j, steady):
        g_wait(t, j)
        s_start(t, j)
        if steady or t >= 2:
            s_wait(t - 2, (j + IA) % NBUF)
        if steady or t + IA < NCHUNK:
            i_start(t + IA, (j + IA) % NBUF)
        if steady or t + GA < NCHUNK:
            i_wait(t + GA, (j + GA) % NBUF)
            g_start(t + GA, (j + GA) % NBUF)

    for t in range(IA):
        i_start(t, t)
    for t in range(GA):
        i_wait(t, t)
        g_start(t, t)

    for t in range(STEADY0):
        step(t, t % NBUF, False)

    def body(k, carry):
        for jj in range(NBUF):
            step(k * NBUF + jj, jj, True)
        return carry

    lax.fori_loop(1, STEADYN // NBUF, body, 0)

    for t in range(STEADYN, NCHUNK):
        step(t, t % NBUF, False)

    for t in range(NCHUNK - 2, NCHUNK):
        s_wait(t, t % NBUF)

    plsc.subcore_barrier()

    pltpu.sync_copy(agg_sh.at[pl.ds(r0, RPT)], agg_hbm.at[c, pl.ds(r0, RPT)])

    @pl.when(s == NS - 1)
    def _write_rem():
        pltpu.sync_copy(agg_sh.at[pl.ds(NS * RPT, REM)],
                        agg_hbm.at[c, pl.ds(NS * RPT, REM)])


@functools.partial(
    pl.kernel, mesh=_SC_MESH,
    out_type=[jax.ShapeDtypeStruct((NC, N, D), jnp.float32)],
    scratch_types=(
        [pltpu.VMEM((CHUNK,), jnp.int32) for _ in range(NBUF)] +
        [pltpu.VMEM((CHUNK, D), jnp.float32)] +
        [pltpu.VMEM_SHARED((N, D), jnp.float32)] +
        [pltpu.SemaphoreType.DMA for _ in range(2 * NBUF)]
    ))
def _sc_cnt(dst_hbm, zer_hbm, cnt_hbm, *scr):
    c = lax.axis_index("c")
    s = lax.axis_index("s")
    didx = list(scr[0:NBUF])
    ones_v = scr[NBUF]
    cnt_sh = scr[NBUF + 1]
    isem = list(scr[NBUF + 2:2 * NBUF + 2])
    ssem = list(scr[2 * NBUF + 2:3 * NBUF + 2])
    wid = c * NS + s
    ebase = wid * EPT

    def i_start(t, j):
        off = pl.multiple_of(ebase + t * CHUNK, 8)
        pltpu.async_copy(dst_hbm.at[pl.ds(off, CHUNK)], didx[j], isem[j])

    def i_wait(t, j):
        off = pl.multiple_of(ebase + t * CHUNK, 8)
        pltpu.make_async_copy(dst_hbm.at[pl.ds(off, CHUNK)], didx[j],
                              isem[j]).wait()

    def s_start(t, j):
        pltpu.async_copy(ones_v, cnt_sh.at[didx[j]], ssem[j], add=True)

    def s_wait(t, j):
        pltpu.make_async_copy(ones_v, cnt_sh.at[didx[j]], ssem[j]).wait()

    _fill_f32(ones_v, CHUNK, D, 1.0)
    r0 = s * RPT
    pltpu.sync_copy(zer_hbm, cnt_sh.at[pl.ds(r0, RPT)])

    @pl.when(s == NS - 1)
    def _zero_rem():
        pltpu.sync_copy(zer_hbm.at[pl.ds(0, REM)],
                        cnt_sh.at[pl.ds(NS * RPT, REM)])

    plsc.subcore_barrier()

    def step(t, j, steady):
        i_wait(t, j)
        s_start(t, j)
        if steady or t >= 2:
            s_wait(t - 2, (j + IA) % NBUF)
        if steady or t + IA < NCHUNK:
            i_start(t + IA, (j + IA) % NBUF)

    for t in range(IA):
        i_start(t, t)

    for t in range(STEADY0):
        step(t, t % NBUF, False)

    def body(k, carry):
        for jj in range(NBUF):
            step(k * NBUF + jj, jj, True)
        return carry

    lax.fori_loop(1, STEADYN // NBUF, body, 0)

    for t in range(STEADYN, NCHUNK):
        step(t, t % NBUF, False)

    for t in range(NCHUNK - 2, NCHUNK):
        s_wait(t, t % NBUF)

    plsc.subcore_barrier()

    pltpu.sync_copy(cnt_sh.at[pl.ds(r0, RPT)], cnt_hbm.at[c, pl.ds(r0, RPT)])

    @pl.when(s == NS - 1)
    def _write_rem():
        pltpu.sync_copy(cnt_sh.at[pl.ds(NS * RPT, REM)],
                        cnt_hbm.at[c, pl.ds(NS * RPT, REM)])


def _sage_first_body(agg_ref, cnt_ref, x_ref, wl_ref, wr_ref, b_ref,
                     h_ref, cm_ref):
    a = agg_ref[0] + agg_ref[1]
    cm = jnp.maximum(cnt_ref[0, :, 0:16] + cnt_ref[1, :, 0:16], 1.0)
    mean = a / cm[:, 0:1]
    out = _dot(mean, wl_ref[...]) + _dot(x_ref[...], wr_ref[...]) + b_ref[...]
    h_ref[...] = jnp.maximum(out, 0.0)
    cm_ref[...] = cm


def _sage_body(relu, agg_ref, cm_ref, x_ref, wl_ref, wr_ref, b_ref, o_ref):
    mean = (agg_ref[0] + agg_ref[1]) / cm_ref[:, 0:1]
    out = _dot(mean, wl_ref[...]) + _dot(x_ref[...], wr_ref[...]) + b_ref[...]
    o_ref[...] = jnp.maximum(out, 0.0) if relu else out


def _moe_body(agg_ref, cm_ref, h_ref, wg_ref, ewl_ref, ewr_ref, eb_ref,
              o_ref):
    h = h_ref[...]
    mean = (agg_ref[0] + agg_ref[1]) / cm_ref[:, 0:1]
    logits = _dot(h, wg_ref[...])
    eidx = lax.broadcasted_iota(jnp.int32, logits.shape, 1)
    m1 = jnp.max(logits, axis=1, keepdims=True)
    i1 = jnp.min(jnp.where(logits == m1, eidx, NUM_EXPERTS), axis=1,
                 keepdims=True)
    rest = jnp.where(eidx == i1, -jnp.inf, logits)
    m2 = jnp.max(rest, axis=1, keepdims=True)
    i2 = jnp.min(jnp.where(rest == m2, eidx, NUM_EXPERTS), axis=1,
                 keepdims=True)
    t = jnp.exp(m2 - m1)
    g1 = 1.0 / (1.0 + t)
    g2 = t / (1.0 + t)
    acc = jnp.zeros_like(h)
    for e in range(NUM_EXPERTS):
        ge = jnp.where(i1 == e, g1, 0.0) + jnp.where(i2 == e, g2, 0.0)
        pe = _dot(mean, ewl_ref[e]) + _dot(h, ewr_ref[e]) + eb_ref[e:e + 1, :]
        acc = acc + ge * pe
    o_ref[...] = jnp.maximum(acc, 0.0)


def _row_spec(shape3=False, width=D):
    if shape3:
        return pl.BlockSpec((NC, B, width), lambda i: (0, i, 0))
    return pl.BlockSpec((B, width), lambda i: (i, 0))


def _full_spec(shape):
    nd = len(shape)
    return pl.BlockSpec(shape, lambda i: (0,) * nd)


def _tc_sage_first(agg, cnt, x, Wl, Wr, b):
    return pl.pallas_call(
        _sage_first_body,
        grid=(GRID,),
        in_specs=[
            _row_spec(True), _row_spec(True), _row_spec(),
            _full_spec((D, D)), _full_spec((D, D)), _full_spec((1, D)),
        ],
        out_specs=[_row_spec(), _row_spec(False, 16)],
        out_shape=[jax.ShapeDtypeStruct((N, D), jnp.float32),
                   jax.ShapeDtypeStruct((N, 16), jnp.float32)],
    )(agg, cnt, x, Wl, Wr, b.reshape(1, D))


def _tc_sage(agg, cm, x, Wl, Wr, b, relu):
    return pl.pallas_call(
        functools.partial(_sage_body, relu),
        grid=(GRID,),
        in_specs=[
            _row_spec(True), _row_spec(False, 16), _row_spec(),
            _full_spec((D, D)), _full_spec((D, D)), _full_spec((1, D)),
        ],
        out_specs=_row_spec(),
        out_shape=jax.ShapeDtypeStruct((N, D), jnp.float32),
    )(agg, cm, x, Wl, Wr, b.reshape(1, D))


def _tc_moe(agg, cm, h, w_gate, eWl, eWr, eb):
    return pl.pallas_call(
        _moe_body,
        grid=(GRID,),
        in_specs=[
            _row_spec(True), _row_spec(False, 16), _row_spec(),
            _full_spec((D, NUM_EXPERTS)),
            _full_spec((NUM_EXPERTS, D, D)), _full_spec((NUM_EXPERTS, D, D)),
            _full_spec((NUM_EXPERTS, D)),
        ],
        out_specs=_row_spec(),
        out_shape=jax.ShapeDtypeStruct((N, D), jnp.float32),
    )(agg, cm, h, w_gate, eWl, eWr, eb)


def kernel(x, W0l, W0r, b0, w_gate, eWl, eWr, eb, W2l, W2r, b2, W3l, W3r, b3,
           edge_index):
    src = edge_index[0].astype(jnp.int32)
    dst = edge_index[1].astype(jnp.int32)
    zer = jnp.zeros((RPT, D), jnp.float32)

    (cnt0,) = _sc_cnt(dst, zer)
    (agg0,) = _sc_agg(x, src, dst, zer)
    h1, cm = _tc_sage_first(agg0, cnt0, x, W0l, W0r, b0)
    (agg1,) = _sc_agg(h1, src, dst, zer)
    h2 = _tc_moe(agg1, cm, h1, w_gate, eWl, eWr, eb)
    (agg2,) = _sc_agg(h2, src, dst, zer)
    h3 = _tc_sage(agg2, cm, h2, W2l, W2r, b2, relu=True)
    (agg3,) = _sc_agg(h3, src, dst, zer)
    out = _tc_sage(agg3, cm, h3, W3l, W3r, b3, relu=False)
    return out


# speedup vs baseline: 12.3531x; 1.0831x over previous
import functools

import jax
import jax.numpy as jnp
from jax import lax
from jax.experimental import pallas as pl
from jax.experimental.pallas import tpu as pltpu
from jax.experimental.pallas import tpu_sc as plsc

N = 10000
E = 320000
D = 128
NUM_EXPERTS = 8

NC = 2
NS = 16
CHUNK = 40
EPC = E // NC
EPT = EPC // NS
NCHUNK = EPT // CHUNK
RPT = 624
REM = N - NS * RPT

B = 2000
GRID = N // B

NBUF = 8
IA = 6
GA = 4
STEADY0 = NBUF
STEADYN = ((NCHUNK - IA) // NBUF) * NBUF


def _dot(a, b):
    return jnp.dot(a, b, preferred_element_type=jnp.float32)


def _fill_f32(ref, nrow, ncol, value):
    ngrp = ncol // 16
    def body(i, carry):
        r = i // ngrp
        g = i % ngrp
        ref[r, pl.ds(g * 16, 16)] = jnp.full((16,), value, jnp.float32)
        return carry
    lax.fori_loop(0, nrow * ngrp, body, 0)


_SC_MESH = plsc.VectorSubcoreMesh(core_axis_name="c", subcore_axis_name="s")


@functools.partial(
    pl.kernel, mesh=_SC_MESH,
    out_type=[jax.ShapeDtypeStruct((NC, N, D), jnp.float32)],
    scratch_types=(
        [pltpu.VMEM((CHUNK,), jnp.int32) for _ in range(NBUF)] +
        [pltpu.VMEM((CHUNK,), jnp.int32) for _ in range(NBUF)] +
        [pltpu.VMEM((CHUNK, D), jnp.float32) for _ in range(NBUF)] +
        [pltpu.VMEM_SHARED((N, D), jnp.float32)] +
        [pltpu.SemaphoreType.DMA for _ in range(3 * NBUF)]
    ))
def _sc_agg(h_hbm, src_hbm, dst_hbm, zer_hbm, agg_hbm, *scr):
    c = lax.axis_index("c")
    s = lax.axis_index("s")
    sidx = list(scr[0:NBUF])
    didx = list(scr[NBUF:2 * NBUF])
    rows = list(scr[2 * NBUF:3 * NBUF])
    agg_sh = scr[3 * NBUF]
    isem = list(scr[3 * NBUF + 1:4 * NBUF + 1])
    gsem = list(scr[4 * NBUF + 1:5 * NBUF + 1])
    ssem = list(scr[5 * NBUF + 1:6 * NBUF + 1])
    wid = c * NS + s
    ebase = wid * EPT

    def i_start(t, j):
        off = pl.multiple_of(ebase + t * CHUNK, 8)
        pltpu.async_copy(src_hbm.at[pl.ds(off, CHUNK)], sidx[j], isem[j])
        pltpu.async_copy(dst_hbm.at[pl.ds(off, CHUNK)], didx[j], isem[j])

    def i_wait(t, j):
        off = pl.multiple_of(ebase + t * CHUNK, 8)
        pltpu.make_async_copy(src_hbm.at[pl.ds(off, CHUNK)], sidx[j],
                              isem[j]).wait()
        pltpu.make_async_copy(dst_hbm.at[pl.ds(off, CHUNK)], didx[j],
                              isem[j]).wait()

    def g_start(t, j):
        pltpu.async_copy(h_hbm.at[sidx[j]], rows[j], gsem[j])

    def g_wait(t, j):
        pltpu.make_async_copy(h_hbm.at[sidx[j]], rows[j], gsem[j]).wait()

    def s_start(t, j):
        pltpu.async_copy(rows[j], agg_sh.at[didx[j]], ssem[j], add=True)

    def s_wait(t, j):
        pltpu.make_async_copy(rows[j], agg_sh.at[didx[j]], ssem[j]).wait()

    r0 = s * RPT
    pltpu.sync_copy(zer_hbm, agg_sh.at[pl.ds(r0, RPT)])

    @pl.when(s == NS - 1)
    def _zero_rem():
        pltpu.sync_copy(zer_hbm.at[pl.ds(0, REM)],
                        agg_sh.at[pl.ds(NS * RPT, REM)])

    plsc.subcore_barrier()

    def step(t, j, steady):
        g_wait(t, j)
        s_start(t, j)
        if steady or t >= 2:
            s_wait(t - 2, (j + IA) % NBUF)
        if steady or t + IA < NCHUNK:
            i_start(t + IA, (j + IA) % NBUF)
        if steady or t + GA < NCHUNK:
            i_wait(t + GA, (j + GA) % NBUF)
            g_start(t + GA, (j + GA) % NBUF)

    for t in range(IA):
        i_start(t, t)
    for t in range(GA):
        i_wait(t, t)
        g_start(t, t)

    for t in range(STEADY0):
        step(t, t % NBUF, False)

    def body(k, carry):
        for jj in range(NBUF):
            step(k * NBUF + jj, jj, True)
        return carry

    lax.fori_loop(1, STEADYN // NBUF, body, 0)

    for t in range(STEADYN, NCHUNK):
        step(t, t % NBUF, False)

    for t in range(NCHUNK - 2, NCHUNK):
        s_wait(t, t % NBUF)

    plsc.subcore_barrier()

    pltpu.sync_copy(agg_sh.at[pl.ds(r0, RPT)], agg_hbm.at[c, pl.ds(r0, RPT)])

    @pl.when(s == NS - 1)
    def _write_rem():
        pltpu.sync_copy(agg_sh.at[pl.ds(NS * RPT, REM)],
                        agg_hbm.at[c, pl.ds(NS * RPT, REM)])


NPAD = 10240
SEG = NPAD // NS


@functools.partial(
    pl.kernel, mesh=_SC_MESH,
    compiler_params=pltpu.CompilerParams(needs_layout_passes=False),
    out_type=[jax.ShapeDtypeStruct((NC * NPAD,), jnp.float32)],
    scratch_types=(
        [pltpu.VMEM((48,), jnp.int32) for _ in range(NBUF)] +
        [pltpu.VMEM((NPAD,), jnp.float32)] +
        [pltpu.VMEM((NS, SEG), jnp.float32)] +
        [pltpu.VMEM((SEG,), jnp.float32)] +
        [pltpu.VMEM_SHARED((NS, 1, NPAD), jnp.float32)] +
        [pltpu.SemaphoreType.DMA for _ in range(NBUF)]
    ))
def _sc_cnt(dst_hbm, cnt_hbm, *scr):
    c = lax.axis_index("c")
    s = lax.axis_index("s")
    didx = list(scr[0:NBUF])
    cntv = scr[NBUF]
    redv = scr[NBUF + 1]
    outv = scr[NBUF + 2]
    stage = scr[NBUF + 3]
    isem = list(scr[NBUF + 4:2 * NBUF + 4])
    wid = c * NS + s
    ebase = wid * EPT

    def i_start(t, j):
        off = pl.multiple_of(ebase + t * CHUNK, 8)
        pltpu.async_copy(dst_hbm.at[pl.ds(off, CHUNK)],
                         didx[j].at[pl.ds(0, CHUNK)], isem[j])

    def i_wait(t, j):
        off = pl.multiple_of(ebase + t * CHUNK, 8)
        pltpu.make_async_copy(dst_hbm.at[pl.ds(off, CHUNK)],
                              didx[j].at[pl.ds(0, CHUNK)], isem[j]).wait()

    def zbody(i, carry):
        cntv[pl.ds(i * 16, 16)] = jnp.zeros((16,), jnp.float32)
        return carry
    lax.fori_loop(0, NPAD // 16, zbody, 0)

    ones16 = jnp.ones((16,), jnp.float32)
    lane = lax.broadcasted_iota(jnp.int32, (16,), 0)

    def addgrp(j, off, tail):
        dv = didx[j][pl.ds(off, 16)]
        if tail:
            dv = jnp.where(lane < 8, dv, NPAD - 1)
        plsc.addupdate_scatter(cntv, [dv], ones16)

    def step(t, j, steady):
        i_wait(t, j)
        addgrp(j, 0, False)
        addgrp(j, 16, False)
        addgrp(j, 32, True)
        if steady or t + IA < NCHUNK:
            i_start(t + IA, (j + IA) % NBUF)

    for t in range(IA):
        i_start(t, t)

    for t in range(STEADY0):
        step(t, t % NBUF, False)

    def body(k, carry):
        for jj in range(NBUF):
            step(k * NBUF + jj, jj, True)
        return carry

    lax.fori_loop(1, STEADYN // NBUF, body, 0)

    for t in range(STEADYN, NCHUNK):
        step(t, t % NBUF, False)

    pltpu.sync_copy(cntv, stage.at[s, 0])
    plsc.subcore_barrier()

    col = s * SEG
    for r in range(NS):
        pltpu.sync_copy(stage.at[r, 0, pl.ds(col, SEG)], redv.at[r])

    def rbody(g, carry):
        acc = redv[0, pl.ds(g * 16, 16)]
        for r in range(1, NS):
            acc = acc + redv[r, pl.ds(g * 16, 16)]
        outv[pl.ds(g * 16, 16)] = acc
        return carry
    lax.fori_loop(0, SEG // 16, rbody, 0)

    pltpu.sync_copy(outv, cnt_hbm.at[pl.ds(c * NPAD + col, SEG)])


def _sage_first_body(agg_ref, cnt_ref, x_ref, wl_ref, wr_ref, b_ref,
                     h_ref, cm_ref):
    a = agg_ref[0] + agg_ref[1]
    cm1 = jnp.maximum(cnt_ref[:, 0:1] + cnt_ref[:, 1:2], 1.0)
    cm = jnp.broadcast_to(cm1, (cm1.shape[0], 16))
    mean = a / cm1
    out = _dot(mean, wl_ref[...]) + _dot(x_ref[...], wr_ref[...]) + b_ref[...]
    h_ref[...] = jnp.maximum(out, 0.0)
    cm_ref[...] = cm


def _sage_body(relu, agg_ref, cm_ref, x_ref, wl_ref, wr_ref, b_ref, o_ref):
    mean = (agg_ref[0] + agg_ref[1]) / cm_ref[:, 0:1]
    out = _dot(mean, wl_ref[...]) + _dot(x_ref[...], wr_ref[...]) + b_ref[...]
    o_ref[...] = jnp.maximum(out, 0.0) if relu else out


def _moe_body(agg_ref, cm_ref, h_ref, wg_ref, ewl_ref, ewr_ref, eb_ref,
              o_ref):
    h = h_ref[...]
    mean = (agg_ref[0] + agg_ref[1]) / cm_ref[:, 0:1]
    logits = _dot(h, wg_ref[...])
    eidx = lax.broadcasted_iota(jnp.int32, logits.shape, 1)
    m1 = jnp.max(logits, axis=1, keepdims=True)
    i1 = jnp.min(jnp.where(logits == m1, eidx, NUM_EXPERTS), axis=1,
                 keepdims=True)
    rest = jnp.where(eidx == i1, -jnp.inf, logits)
    m2 = jnp.max(rest, axis=1, keepdims=True)
    i2 = jnp.min(jnp.where(rest == m2, eidx, NUM_EXPERTS), axis=1,
                 keepdims=True)
    t = jnp.exp(m2 - m1)
    g1 = 1.0 / (1.0 + t)
    g2 = t / (1.0 + t)
    acc = jnp.zeros_like(h)
    for e in range(NUM_EXPERTS):
        ge = jnp.where(i1 == e, g1, 0.0) + jnp.where(i2 == e, g2, 0.0)
        pe = _dot(mean, ewl_ref[e]) + _dot(h, ewr_ref[e]) + eb_ref[e:e + 1, :]
        acc = acc + ge * pe
    o_ref[...] = jnp.maximum(acc, 0.0)


def _row_spec(shape3=False, width=D):
    if shape3:
        return pl.BlockSpec((NC, B, width), lambda i: (0, i, 0))
    return pl.BlockSpec((B, width), lambda i: (i, 0))


def _full_spec(shape):
    nd = len(shape)
    return pl.BlockSpec(shape, lambda i: (0,) * nd)


def _tc_sage_first(agg, cnt, x, Wl, Wr, b):
    return pl.pallas_call(
        _sage_first_body,
        grid=(GRID,),
        in_specs=[
            _row_spec(True), _row_spec(False, 2), _row_spec(),
            _full_spec((D, D)), _full_spec((D, D)), _full_spec((1, D)),
        ],
        out_specs=[_row_spec(), _row_spec(False, 16)],
        out_shape=[jax.ShapeDtypeStruct((N, D), jnp.float32),
                   jax.ShapeDtypeStruct((N, 16), jnp.float32)],
    )(agg, cnt, x, Wl, Wr, b.reshape(1, D))


def _tc_sage(agg, cm, x, Wl, Wr, b, relu):
    return pl.pallas_call(
        functools.partial(_sage_body, relu),
        grid=(GRID,),
        in_specs=[
            _row_spec(True), _row_spec(False, 16), _row_spec(),
            _full_spec((D, D)), _full_spec((D, D)), _full_spec((1, D)),
        ],
        out_specs=_row_spec(),
        out_shape=jax.ShapeDtypeStruct((N, D), jnp.float32),
    )(agg, cm, x, Wl, Wr, b.reshape(1, D))


def _tc_moe(agg, cm, h, w_gate, eWl, eWr, eb):
    return pl.pallas_call(
        _moe_body,
        grid=(GRID,),
        in_specs=[
            _row_spec(True), _row_spec(False, 16), _row_spec(),
            _full_spec((D, NUM_EXPERTS)),
            _full_spec((NUM_EXPERTS, D, D)), _full_spec((NUM_EXPERTS, D, D)),
            _full_spec((NUM_EXPERTS, D)),
        ],
        out_specs=_row_spec(),
        out_shape=jax.ShapeDtypeStruct((N, D), jnp.float32),
    )(agg, cm, h, w_gate, eWl, eWr, eb)


def kernel(x, W0l, W0r, b0, w_gate, eWl, eWr, eb, W2l, W2r, b2, W3l, W3r, b3,
           edge_index):
    src = edge_index[0].astype(jnp.int32)
    dst = edge_index[1].astype(jnp.int32)
    zer = jnp.zeros((RPT, D), jnp.float32)

    (cnt1d,) = _sc_cnt(dst)
    cnt0 = cnt1d.reshape(NC, NPAD)[:, :N].T
    (agg0,) = _sc_agg(x, src, dst, zer)
    h1, cm = _tc_sage_first(agg0, cnt0, x, W0l, W0r, b0)
    (agg1,) = _sc_agg(h1, src, dst, zer)
    h2 = _tc_moe(agg1, cm, h1, w_gate, eWl, eWr, eb)
    (agg2,) = _sc_agg(h2, src, dst, zer)
    h3 = _tc_sage(agg2, cm, h2, W2l, W2r, b2, relu=True)
    (agg3,) = _sc_agg(h3, src, dst, zer)
    out = _tc_sage(agg3, cm, h3, W3l, W3r, b3, relu=False)
    return out


# speedup vs baseline: 12.4317x; 1.0064x over previous
import functools

import jax
import jax.numpy as jnp
from jax import lax
from jax.experimental import pallas as pl
from jax.experimental.pallas import tpu as pltpu
from jax.experimental.pallas import tpu_sc as plsc

N = 10000
E = 320000
D = 128
NUM_EXPERTS = 8

NC = 2
NS = 16
CHUNK = 40
EPC = E // NC
EPT = EPC // NS
NCHUNK = EPT // CHUNK
RPT = 624
REM = N - NS * RPT

B = 2000
GRID = N // B

NBUF = 8
IA = 6
GA = 4
STEADY0 = NBUF
STEADYN = ((NCHUNK - IA) // NBUF) * NBUF


def _dot(a, b):
    return jnp.dot(a, b, preferred_element_type=jnp.float32)


def _fill_f32(ref, nrow, ncol, value):
    ngrp = ncol // 16
    def body(i, carry):
        r = i // ngrp
        g = i % ngrp
        ref[r, pl.ds(g * 16, 16)] = jnp.full((16,), value, jnp.float32)
        return carry
    lax.fori_loop(0, nrow * ngrp, body, 0)


_SC_MESH = plsc.VectorSubcoreMesh(core_axis_name="c", subcore_axis_name="s")


@functools.partial(
    pl.kernel, mesh=_SC_MESH,
    out_type=[jax.ShapeDtypeStruct((NC, N, D), jnp.float32)],
    scratch_types=(
        [pltpu.VMEM((CHUNK,), jnp.int32) for _ in range(NBUF)] +
        [pltpu.VMEM((CHUNK,), jnp.int32) for _ in range(NBUF)] +
        [pltpu.VMEM((CHUNK, D), jnp.float32) for _ in range(NBUF)] +
        [pltpu.VMEM_SHARED((N, D), jnp.float32)] +
        [pltpu.SemaphoreType.DMA for _ in range(3 * NBUF)]
    ))
def _sc_agg(h_hbm, src_hbm, dst_hbm, zer_hbm, agg_hbm, *scr):
    c = lax.axis_index("c")
    s = lax.axis_index("s")
    sidx = list(scr[0:NBUF])
    didx = list(scr[NBUF:2 * NBUF])
    rows = list(scr[2 * NBUF:3 * NBUF])
    agg_sh = scr[3 * NBUF]
    isem = list(scr[3 * NBUF + 1:4 * NBUF + 1])
    gsem = list(scr[4 * NBUF + 1:5 * NBUF + 1])
    ssem = list(scr[5 * NBUF + 1:6 * NBUF + 1])
    wid = c * NS + s
    ebase = wid * EPT

    def i_start(t, j):
        off = pl.multiple_of(ebase + t * CHUNK, 8)
        pltpu.async_copy(src_hbm.at[pl.ds(off, CHUNK)], sidx[j], isem[j])
        pltpu.async_copy(dst_hbm.at[pl.ds(off, CHUNK)], didx[j], isem[j])

    def i_wait(t, j):
        off = pl.multiple_of(ebase + t * CHUNK, 8)
        pltpu.make_async_copy(src_hbm.at[pl.ds(off, CHUNK)], sidx[j],
                              isem[j]).wait()
        pltpu.make_async_copy(dst_hbm.at[pl.ds(off, CHUNK)], didx[j],
                              isem[j]).wait()

    def g_start(t, j):
        pltpu.async_copy(h_hbm.at[sidx[j]], rows[j], gsem[j])

    def g_wait(t, j):
        pltpu.make_async_copy(h_hbm.at[sidx[j]], rows[j], gsem[j]).wait()

    def s_start(t, j):
        pltpu.async_copy(rows[j], agg_sh.at[didx[j]], ssem[j], add=True)

    def s_wait(t, j):
        pltpu.make_async_copy(rows[j], agg_sh.at[didx[j]], ssem[j]).wait()

    r0 = s * RPT
    pltpu.sync_copy(zer_hbm, agg_sh.at[pl.ds(r0, RPT)])

    @pl.when(s == NS - 1)
    def _zero_rem():
        pltpu.sync_copy(zer_hbm.at[pl.ds(0, REM)],
                        agg_sh.at[pl.ds(NS * RPT, REM)])

    plsc.subcore_barrier()

    def step(t, j, steady):
        g_wait(t, j)
        s_start(t, j)
        if steady or t >= 2:
            s_wait(t - 2, (j + IA) % NBUF)
        if steady or t + IA < NCHUNK:
            i_start(t + IA, (j + IA) % NBUF)
        if steady or t + GA < NCHUNK:
            i_wait(t + GA, (j + GA) % NBUF)
            g_start(t + GA, (j + GA) % NBUF)

    for t in range(IA):
        i_start(t, t)
    for t in range(GA):
        i_wait(t, t)
        g_start(t, t)

    for t in range(STEADY0):
        step(t, t % NBUF, False)

    def body(k, carry):
        for jj in range(NBUF):
            step(k * NBUF + jj, jj, True)
        return carry

    lax.fori_loop(1, STEADYN // NBUF, body, 0)

    for t in range(STEADYN, NCHUNK):
        step(t, t % NBUF, False)

    for t in range(NCHUNK - 2, NCHUNK):
        s_wait(t, t % NBUF)

    plsc.subcore_barrier()

    pltpu.sync_copy(agg_sh.at[pl.ds(r0, RPT)], agg_hbm.at[c, pl.ds(r0, RPT)])

    @pl.when(s == NS - 1)
    def _write_rem():
        pltpu.sync_copy(agg_sh.at[pl.ds(NS * RPT, REM)],
                        agg_hbm.at[c, pl.ds(NS * RPT, REM)])


NPAD = 10240
SEG = NPAD // NS


@functools.partial(
    pl.kernel, mesh=_SC_MESH,
    compiler_params=pltpu.CompilerParams(needs_layout_passes=False),
    out_type=[jax.ShapeDtypeStruct((NC * NPAD,), jnp.float32)],
    scratch_types=(
        [pltpu.VMEM((48,), jnp.int32) for _ in range(NBUF)] +
        [pltpu.VMEM((NPAD,), jnp.float32)] +
        [pltpu.VMEM((NS, SEG), jnp.float32)] +
        [pltpu.VMEM((SEG,), jnp.float32)] +
        [pltpu.VMEM_SHARED((NS, 1, NPAD), jnp.float32)] +
        [pltpu.SemaphoreType.DMA for _ in range(NBUF)]
    ))
def _sc_cnt(dst_hbm, cnt_hbm, *scr):
    c = lax.axis_index("c")
    s = lax.axis_index("s")
    didx = list(scr[0:NBUF])
    cntv = scr[NBUF]
    redv = scr[NBUF + 1]
    outv = scr[NBUF + 2]
    stage = scr[NBUF + 3]
    isem = list(scr[NBUF + 4:2 * NBUF + 4])
    wid = c * NS + s
    ebase = wid * EPT

    def i_start(t, j):
        off = pl.multiple_of(ebase + t * CHUNK, 8)
        pltpu.async_copy(dst_hbm.at[pl.ds(off, CHUNK)],
                         didx[j].at[pl.ds(0, CHUNK)], isem[j])

    def i_wait(t, j):
        off = pl.multiple_of(ebase + t * CHUNK, 8)
        pltpu.make_async_copy(dst_hbm.at[pl.ds(off, CHUNK)],
                              didx[j].at[pl.ds(0, CHUNK)], isem[j]).wait()

    def zbody(i, carry):
        cntv[pl.ds(i * 16, 16)] = jnp.zeros((16,), jnp.float32)
        return carry
    lax.fori_loop(0, NPAD // 16, zbody, 0)

    ones16 = jnp.ones((16,), jnp.float32)
    lane = lax.broadcasted_iota(jnp.int32, (16,), 0)

    def addgrp(j, off, tail):
        dv = didx[j][pl.ds(off, 16)]
        if tail:
            dv = jnp.where(lane < 8, dv, NPAD - 1)
        plsc.addupdate_scatter(cntv, [dv], ones16)

    def step(t, j, steady):
        i_wait(t, j)
        addgrp(j, 0, False)
        addgrp(j, 16, False)
        addgrp(j, 32, True)
        if steady or t + IA < NCHUNK:
            i_start(t + IA, (j + IA) % NBUF)

    for t in range(IA):
        i_start(t, t)

    for t in range(STEADY0):
        step(t, t % NBUF, False)

    def body(k, carry):
        for jj in range(NBUF):
            step(k * NBUF + jj, jj, True)
        return carry

    lax.fori_loop(1, STEADYN // NBUF, body, 0)

    for t in range(STEADYN, NCHUNK):
        step(t, t % NBUF, False)

    pltpu.sync_copy(cntv, stage.at[s, 0])
    plsc.subcore_barrier()

    col = s * SEG
    for r in range(NS):
        pltpu.sync_copy(stage.at[r, 0, pl.ds(col, SEG)], redv.at[r])

    def rbody(g, carry):
        acc = redv[0, pl.ds(g * 16, 16)]
        for r in range(1, NS):
            acc = acc + redv[r, pl.ds(g * 16, 16)]
        outv[pl.ds(g * 16, 16)] = acc
        return carry
    lax.fori_loop(0, SEG // 16, rbody, 0)

    pltpu.sync_copy(outv, cnt_hbm.at[pl.ds(c * NPAD + col, SEG)])


def _self_body(x_ref, wr_ref, b_ref, o_ref):
    o_ref[...] = _dot(x_ref[...], wr_ref[...]) + b_ref[...]


def _combine_first_body(agg_ref, cnt_ref, s_ref, wl_ref, h_ref, cm_ref):
    a = agg_ref[0] + agg_ref[1]
    cm1 = jnp.maximum(cnt_ref[:, 0:1] + cnt_ref[:, 1:2], 1.0)
    cm = jnp.broadcast_to(cm1, (cm1.shape[0], 16))
    mean = a / cm1
    h_ref[...] = jnp.maximum(_dot(mean, wl_ref[...]) + s_ref[...], 0.0)
    cm_ref[...] = cm


def _combine_body(relu, agg_ref, cm_ref, s_ref, wl_ref, o_ref):
    mean = (agg_ref[0] + agg_ref[1]) / cm_ref[:, 0:1]
    out = _dot(mean, wl_ref[...]) + s_ref[...]
    o_ref[...] = jnp.maximum(out, 0.0) if relu else out


def _moe_self_body(h_ref, wg_ref, ewr_ref, eb_ref, s_ref, gates_ref):
    h = h_ref[...]
    logits = _dot(h, wg_ref[...])
    eidx = lax.broadcasted_iota(jnp.int32, logits.shape, 1)
    m1 = jnp.max(logits, axis=1, keepdims=True)
    i1 = jnp.min(jnp.where(logits == m1, eidx, NUM_EXPERTS), axis=1,
                 keepdims=True)
    rest = jnp.where(eidx == i1, -jnp.inf, logits)
    m2 = jnp.max(rest, axis=1, keepdims=True)
    i2 = jnp.min(jnp.where(rest == m2, eidx, NUM_EXPERTS), axis=1,
                 keepdims=True)
    t = jnp.exp(m2 - m1)
    g1 = 1.0 / (1.0 + t)
    g2 = t / (1.0 + t)
    gates = (jnp.where(eidx == i1, g1, 0.0) +
             jnp.where(eidx == i2, g2, 0.0))
    acc = jnp.zeros_like(h)
    for e in range(NUM_EXPERTS):
        acc = acc + gates[:, e:e + 1] * (_dot(h, ewr_ref[e]) +
                                         eb_ref[e:e + 1, :])
    s_ref[...] = acc
    gates_ref[...] = gates


def _moe_combine_body(agg_ref, cm_ref, s_ref, gates_ref, ewl_ref, o_ref):
    mean = (agg_ref[0] + agg_ref[1]) / cm_ref[:, 0:1]
    acc = s_ref[...]
    for e in range(NUM_EXPERTS):
        acc = acc + gates_ref[:, e:e + 1] * _dot(mean, ewl_ref[e])
    o_ref[...] = jnp.maximum(acc, 0.0)


def _row_spec(shape3=False, width=D):
    if shape3:
        return pl.BlockSpec((NC, B, width), lambda i: (0, i, 0))
    return pl.BlockSpec((B, width), lambda i: (i, 0))


def _full_spec(shape):
    nd = len(shape)
    return pl.BlockSpec(shape, lambda i: (0,) * nd)


def _tc_self(h, Wr, b):
    return pl.pallas_call(
        _self_body,
        grid=(GRID,),
        in_specs=[_row_spec(), _full_spec((D, D)), _full_spec((1, D))],
        out_specs=_row_spec(),
        out_shape=jax.ShapeDtypeStruct((N, D), jnp.float32),
    )(h, Wr, b.reshape(1, D))


def _tc_combine_first(agg, cnt, S, Wl):
    return pl.pallas_call(
        _combine_first_body,
        grid=(GRID,),
        in_specs=[
            _row_spec(True), _row_spec(False, 2), _row_spec(),
            _full_spec((D, D)),
        ],
        out_specs=[_row_spec(), _row_spec(False, 16)],
        out_shape=[jax.ShapeDtypeStruct((N, D), jnp.float32),
                   jax.ShapeDtypeStruct((N, 16), jnp.float32)],
    )(agg, cnt, S, Wl)


def _tc_combine(agg, cm, S, Wl, relu):
    return pl.pallas_call(
        functools.partial(_combine_body, relu),
        grid=(GRID,),
        in_specs=[
            _row_spec(True), _row_spec(False, 16), _row_spec(),
            _full_spec((D, D)),
        ],
        out_specs=_row_spec(),
        out_shape=jax.ShapeDtypeStruct((N, D), jnp.float32),
    )(agg, cm, S, Wl)


def _tc_moe_self(h, w_gate, eWr, eb):
    return pl.pallas_call(
        _moe_self_body,
        grid=(GRID,),
        in_specs=[
            _row_spec(), _full_spec((D, NUM_EXPERTS)),
            _full_spec((NUM_EXPERTS, D, D)), _full_spec((NUM_EXPERTS, D)),
        ],
        out_specs=[_row_spec(), _row_spec(False, NUM_EXPERTS)],
        out_shape=[jax.ShapeDtypeStruct((N, D), jnp.float32),
                   jax.ShapeDtypeStruct((N, NUM_EXPERTS), jnp.float32)],
    )(h, w_gate, eWr, eb)


def _tc_moe_combine(agg, cm, S, gates, eWl):
    return pl.pallas_call(
        _moe_combine_body,
        grid=(GRID,),
        in_specs=[
            _row_spec(True), _row_spec(False, 16), _row_spec(),
            _row_spec(False, NUM_EXPERTS),
            _full_spec((NUM_EXPERTS, D, D)),
        ],
        out_specs=_row_spec(),
        out_shape=jax.ShapeDtypeStruct((N, D), jnp.float32),
    )(agg, cm, S, gates, eWl)


def kernel(x, W0l, W0r, b0, w_gate, eWl, eWr, eb, W2l, W2r, b2, W3l, W3r, b3,
           edge_index):
    src = edge_index[0].astype(jnp.int32)
    dst = edge_index[1].astype(jnp.int32)
    zer = jnp.zeros((RPT, D), jnp.float32)

    (cnt1d,) = _sc_cnt(dst)
    cnt0 = cnt1d.reshape(NC, NPAD)[:, :N].T
    (agg0,) = _sc_agg(x, src, dst, zer)
    S1 = _tc_self(x, W0r, b0)
    h1, cm = _tc_combine_first(agg0, cnt0, S1, W0l)

    (agg1,) = _sc_agg(h1, src, dst, zer)
    S2, gates = _tc_moe_self(h1, w_gate, eWr, eb)
    h2 = _tc_moe_combine(agg1, cm, S2, gates, eWl)

    (agg2,) = _sc_agg(h2, src, dst, zer)
    S3 = _tc_self(h2, W2r, b2)
    h3 = _tc_combine(agg2, cm, S3, W2l, relu=True)

    (agg3,) = _sc_agg(h3, src, dst, zer)
    S4 = _tc_self(h3, W3r, b3)
    out = _tc_combine(agg3, cm, S4, W3l, relu=False)
    return out
